# baseline (device time: 186750 ns/iter reference)
import jax
import jax.numpy as jnp
from jax import lax
from jax.experimental import pallas as pl
from jax.experimental.pallas import tpu as pltpu

N_DEV = 4
B_PER = 2
SQ = 512
SKV = 512
HG = 32
H_PER = 8
DH = 64
D_MODEL = 768
SCALE = 0.125


def kernel(x, Wq, K_ext, V_ext, Wo):
    x = (x * SCALE).astype(jnp.bfloat16)
    Wq = Wq.astype(jnp.bfloat16)
    Wo = Wo.astype(jnp.bfloat16)

    def body(x_ref, wq_ref, k_hbm, v_hbm, wo_ref, out_ref,
             wq_comm, wo_comm, ctx_ref, k_buf, v_buf,
             send_q, recv_q, send_o, recv_o, k_sems, v_sems):
        my_pos = lax.axis_index("i")

        def start_kv_dma(kslot, origin):
            descs = []
            for b in range(B_PER):
                bg = my_pos * B_PER + b
                for h in range(H_PER):
                    head = origin * H_PER + h
                    dk = pltpu.make_async_copy(
                        k_hbm.at[bg, :, head], k_buf.at[kslot, b, h],
                        k_sems.at[kslot, b, h],
                    )
                    dv = pltpu.make_async_copy(
                        v_hbm.at[bg, :, head], v_buf.at[kslot, b, h],
                        v_sems.at[kslot, b, h],
                    )
                    dk.start()
                    dv.start()
                    descs += [dk, dv]
            return descs

        kv_own = start_kv_dma(0, my_pos)

        barrier_sem = pltpu.get_barrier_semaphore()
        for d in range(1, N_DEV):
            peer = lax.rem(my_pos + d, N_DEV)
            pl.semaphore_signal(
                barrier_sem, inc=1,
                device_id=(peer,), device_id_type=pl.DeviceIdType.MESH,
            )
        pl.semaphore_wait(barrier_sem, N_DEV - 1)

        sends = []
        for d in range(1, N_DEV):
            target = lax.rem(my_pos + d, N_DEV)
            slot = N_DEV - 1 - d
            rq = pltpu.make_async_remote_copy(
                src_ref=wq_ref, dst_ref=wq_comm.at[slot],
                send_sem=send_q.at[d - 1], recv_sem=recv_q.at[slot],
                device_id=(target,), device_id_type=pl.DeviceIdType.MESH,
            )
            ro = pltpu.make_async_remote_copy(
                src_ref=wo_ref, dst_ref=wo_comm.at[slot],
                send_sem=send_o.at[d - 1], recv_sem=recv_o.at[slot],
                device_id=(target,), device_id_type=pl.DeviceIdType.MESH,
            )
            rq.start()
            ro.start()
            sends.append((rq, ro))

        kv_next = start_kv_dma(1, lax.rem(my_pos + N_DEV - 1, N_DEV))

        qi = lax.broadcasted_iota(jnp.int32, (SQ, SKV), 0)
        ki = lax.broadcasted_iota(jnp.int32, (SQ, SKV), 1)
        mask = (jnp.abs(qi - ki) <= 128) | (ki < 32) | (qi < 32)
        bias = jnp.where(mask, 0.0, -1e9).astype(jnp.bfloat16)

        def compute_chunk(wq_c, wo_c, kslot, first):
            for b in range(B_PER):
                q = lax.dot_general(
                    x_ref[b], wq_c, (((1,), (0,)), ((), ())),
                    preferred_element_type=jnp.float32,
                ).astype(jnp.bfloat16)
                for h in range(H_PER):
                    qh = q[:, h * DH:(h + 1) * DH]
                    kh = k_buf[kslot, b, h].astype(jnp.bfloat16)
                    s = lax.dot_general(
                        qh, kh, (((1,), (1,)), ((), ())),
                        preferred_element_type=jnp.float32,
                    ).astype(jnp.bfloat16)
                    w = jnp.exp(s + bias)
                    wsum = jnp.sum(w, axis=-1, keepdims=True,
                                   dtype=jnp.float32)
                    vh = v_buf[kslot, b, h].astype(jnp.bfloat16)
                    ctx_h = lax.dot_general(
                        w, vh, (((1,), (0,)), ((), ())),
                        preferred_element_type=jnp.float32,
                    )
                    ctx_ref[:, h * DH:(h + 1) * DH] = (ctx_h / wsum).astype(jnp.bfloat16)
                partial = lax.dot_general(
                    ctx_ref[...], wo_c, (((1,), (0,)), ((), ())),
                    preferred_element_type=jnp.float32,
                )
                if first:
                    out_ref[b] = partial
                else:
                    out_ref[b] = out_ref[b] + partial

        for desc in kv_own:
            desc.wait()
        compute_chunk(wq_ref[...], wo_ref[...], 0, first=True)

        kv_orders = [
            (2, 1, lax.rem(my_pos + 1, N_DEV)),
            (0, 0, lax.rem(my_pos + 2, N_DEV)),
            (1, 1, None),
        ]
        for wslot, kslot, prefetch_origin in kv_orders:
            if prefetch_origin is not None:
                kv_next += start_kv_dma(1 - kslot, prefetch_origin)
            recv_desc_q = pltpu.make_async_remote_copy(
                src_ref=wq_ref, dst_ref=wq_comm.at[wslot],
                send_sem=send_q.at[0], recv_sem=recv_q.at[wslot],
                device_id=(my_pos,), device_id_type=pl.DeviceIdType.MESH,
            )
            recv_desc_o = pltpu.make_async_remote_copy(
                src_ref=wo_ref, dst_ref=wo_comm.at[wslot],
                send_sem=send_o.at[0], recv_sem=recv_o.at[wslot],
                device_id=(my_pos,), device_id_type=pl.DeviceIdType.MESH,
            )
            recv_desc_q.wait_recv()
            recv_desc_o.wait_recv()
            n_wait = 32
            for desc in kv_next[:n_wait]:
                desc.wait()
            kv_next = kv_next[n_wait:]
            compute_chunk(wq_comm[wslot], wo_comm[wslot], kslot, first=False)

        for rq, ro in sends:
            rq.wait_send()
            ro.wait_send()

    return pl.pallas_call(
        body,
        out_shape=jax.ShapeDtypeStruct((B_PER, SQ, D_MODEL), jnp.float32),
        in_specs=[
            pl.BlockSpec(memory_space=pltpu.VMEM),
            pl.BlockSpec(memory_space=pltpu.VMEM),
            pl.BlockSpec(memory_space=pl.ANY),
            pl.BlockSpec(memory_space=pl.ANY),
            pl.BlockSpec(memory_space=pltpu.VMEM),
        ],
        out_specs=pl.BlockSpec(memory_space=pltpu.VMEM),
        scratch_shapes=[
            pltpu.VMEM((N_DEV - 1, D_MODEL, H_PER * DH), jnp.bfloat16),
            pltpu.VMEM((N_DEV - 1, H_PER * DH, D_MODEL), jnp.bfloat16),
            pltpu.VMEM((SQ, H_PER * DH), jnp.bfloat16),
            pltpu.VMEM((2, B_PER, H_PER, SKV, DH), jnp.float32),
            pltpu.VMEM((2, B_PER, H_PER, SKV, DH), jnp.float32),
            pltpu.SemaphoreType.DMA((N_DEV - 1,)),
            pltpu.SemaphoreType.DMA((N_DEV - 1,)),
            pltpu.SemaphoreType.DMA((N_DEV - 1,)),
            pltpu.SemaphoreType.DMA((N_DEV - 1,)),
            pltpu.SemaphoreType.DMA((2, B_PER, H_PER)),
            pltpu.SemaphoreType.DMA((2, B_PER, H_PER)),
        ],
        compiler_params=pltpu.CompilerParams(
            collective_id=0,
            vmem_limit_bytes=100 * 1024 * 1024,
        ),
    )(x, Wq, K_ext, V_ext, Wo)


# device time: 87365 ns/iter; 2.1376x vs baseline; 2.1376x over previous
import jax
import jax.numpy as jnp
from jax import lax
from jax.experimental import pallas as pl
from jax.experimental.pallas import tpu as pltpu

N_DEV = 4
B_PER = 2
SQ = 512
SKV = 512
HG = 32
H_PER = 8
DH = 64
D_MODEL = 768
SCALE = 0.125


def kernel(x, Wq, K_ext, V_ext, Wo):
    my = lax.axis_index("i")

    k_my = lax.dynamic_slice(K_ext, (my * B_PER, 0, 0, 0), (B_PER, SKV, HG, DH))
    v_my = lax.dynamic_slice(V_ext, (my * B_PER, 0, 0, 0), (B_PER, SKV, HG, DH))
    k_r = k_my.reshape(B_PER, SKV, N_DEV, H_PER, DH).transpose(2, 0, 3, 1, 4)
    v_r = v_my.reshape(B_PER, SKV, N_DEV, H_PER, DH).transpose(2, 0, 3, 1, 4)
    x = (x * SCALE).astype(jnp.bfloat16)
    k_r = k_r.astype(jnp.bfloat16)
    v_r = v_r.astype(jnp.bfloat16)
    Wq = Wq.astype(jnp.bfloat16)
    Wo = Wo.astype(jnp.bfloat16)

    def body(x_ref, wq_ref, k_ref, v_ref, wo_ref, out_ref,
             wq_comm, wo_comm, ctx_ref, send_q, recv_q, send_o, recv_o):
        my_pos = lax.axis_index("i")

        barrier_sem = pltpu.get_barrier_semaphore()
        for d in range(1, N_DEV):
            peer = lax.rem(my_pos + d, N_DEV)
            pl.semaphore_signal(
                barrier_sem, inc=1,
                device_id=(peer,), device_id_type=pl.DeviceIdType.MESH,
            )
        pl.semaphore_wait(barrier_sem, N_DEV - 1)

        HH = H_PER * DH // 2
        sends = []
        for half in range(2):
            cs = slice(half * HH, (half + 1) * HH)
            for d in range(1, N_DEV):
                target = lax.rem(my_pos + d, N_DEV)
                slot = N_DEV - 1 - d
                rq = pltpu.make_async_remote_copy(
                    src_ref=wq_ref.at[:, cs], dst_ref=wq_comm.at[slot, :, cs],
                    send_sem=send_q.at[d - 1, half], recv_sem=recv_q.at[slot, half],
                    device_id=(target,), device_id_type=pl.DeviceIdType.MESH,
                )
                ro = pltpu.make_async_remote_copy(
                    src_ref=wo_ref.at[cs], dst_ref=wo_comm.at[slot, cs],
                    send_sem=send_o.at[d - 1, half], recv_sem=recv_o.at[slot, half],
                    device_id=(target,), device_id_type=pl.DeviceIdType.MESH,
                )
                rq.start()
                ro.start()
                sends.append((rq, ro))

        qi = lax.broadcasted_iota(jnp.int32, (SQ, SKV), 0)
        ki = lax.broadcasted_iota(jnp.int32, (SQ, SKV), 1)
        mask = (jnp.abs(qi - ki) <= 128) | (ki < 32) | (qi < 32)
        bias = jnp.where(mask, 0.0, -1e9).astype(jnp.bfloat16)

        def compute_block(wq_c, wo_c, origin, h_lo, n_h, first):
            for b in range(B_PER):
                q = lax.dot_general(
                    x_ref[b], wq_c, (((1,), (0,)), ((), ())),
                    preferred_element_type=jnp.float32,
                ).astype(jnp.bfloat16)
                for h in range(n_h):
                    qh = q[:, h * DH:(h + 1) * DH]
                    kh = k_ref[origin, b, h_lo + h]
                    s = lax.dot_general(
                        qh, kh, (((1,), (1,)), ((), ())),
                        preferred_element_type=jnp.float32,
                    ).astype(jnp.bfloat16)
                    w = jnp.exp(s + bias)
                    wsum = jnp.sum(w, axis=-1, keepdims=True,
                                   dtype=jnp.float32)
                    vh = v_ref[origin, b, h_lo + h]
                    ctx_h = lax.dot_general(
                        w, vh, (((1,), (0,)), ((), ())),
                        preferred_element_type=jnp.float32,
                    )
                    ctx_ref[:, h * DH:(h + 1) * DH] = (ctx_h / wsum).astype(jnp.bfloat16)
                partial = lax.dot_general(
                    ctx_ref[:, :n_h * DH], wo_c, (((1,), (0,)), ((), ())),
                    preferred_element_type=jnp.float32,
                )
                if first:
                    out_ref[b] = partial
                else:
                    out_ref[b] = out_ref[b] + partial

        compute_block(wq_ref[...], wo_ref[...], my_pos, 0, H_PER, first=True)

        for slot, half in ((2, 0), (0, 0), (2, 1), (0, 1), (1, 0), (1, 1)):
            cs = slice(half * HH, (half + 1) * HH)
            recv_desc_q = pltpu.make_async_remote_copy(
                src_ref=wq_ref.at[:, cs], dst_ref=wq_comm.at[slot, :, cs],
                send_sem=send_q.at[0, half], recv_sem=recv_q.at[slot, half],
                device_id=(my_pos,), device_id_type=pl.DeviceIdType.MESH,
            )
            recv_desc_o = pltpu.make_async_remote_copy(
                src_ref=wo_ref.at[cs], dst_ref=wo_comm.at[slot, cs],
                send_sem=send_o.at[0, half], recv_sem=recv_o.at[slot, half],
                device_id=(my_pos,), device_id_type=pl.DeviceIdType.MESH,
            )
            recv_desc_q.wait_recv()
            recv_desc_o.wait_recv()
            origin = lax.rem(my_pos + slot + 1, N_DEV)
            compute_block(wq_comm[slot][:, cs], wo_comm[slot][cs],
                          origin, half * (H_PER // 2), H_PER // 2, first=False)

        for rq, ro in sends:
            rq.wait_send()
            ro.wait_send()

    return pl.pallas_call(
        body,
        out_shape=jax.ShapeDtypeStruct((B_PER, SQ, D_MODEL), jnp.float32),
        in_specs=[
            pl.BlockSpec(memory_space=pltpu.VMEM),
            pl.BlockSpec(memory_space=pltpu.VMEM),
            pl.BlockSpec(memory_space=pltpu.VMEM),
            pl.BlockSpec(memory_space=pltpu.VMEM),
            pl.BlockSpec(memory_space=pltpu.VMEM),
        ],
        out_specs=pl.BlockSpec(memory_space=pltpu.VMEM),
        scratch_shapes=[
            pltpu.VMEM((N_DEV - 1, D_MODEL, H_PER * DH), jnp.bfloat16),
            pltpu.VMEM((N_DEV - 1, H_PER * DH, D_MODEL), jnp.bfloat16),
            pltpu.VMEM((SQ, H_PER * DH), jnp.bfloat16),
            pltpu.SemaphoreType.DMA((N_DEV - 1, 2)),
            pltpu.SemaphoreType.DMA((N_DEV - 1, 2)),
            pltpu.SemaphoreType.DMA((N_DEV - 1, 2)),
            pltpu.SemaphoreType.DMA((N_DEV - 1, 2)),
        ],
        compiler_params=pltpu.CompilerParams(
            collective_id=0,
            vmem_limit_bytes=100 * 1024 * 1024,
        ),
    )(x, Wq, k_r, v_r, Wo)
